# baseline (speedup 1.0000x reference)
import functools

import jax
import jax.numpy as jnp
from jax import lax
from jax.experimental import pallas as pl
from jax.experimental.pallas import tpu as pltpu
from jax.experimental.pallas import tpu_sc as plsc

NC = 2
NS = 16
LANES = 16
CHUNK = 128
ROWS_PER_TILE = 640

_F32 = jnp.float32
_I32 = jnp.int32


def _zero_rows(zb, d):
    z = jnp.zeros((LANES,), _F32)

    def body(i, _):
        for k in range(d // LANES):
            zb[i, pl.ds(k * LANES, LANES)] = z
        return 0

    lax.fori_loop(0, 128, body, 0)


def _make_sc_deg(n_pad, n_chunk_rows):
    seg = n_pad // NS

    @functools.partial(
        pl.kernel,
        mesh=plsc.VectorSubcoreMesh(core_axis_name="c", subcore_axis_name="s"),
        out_type=jax.ShapeDtypeStruct((n_pad,), _F32),
        compiler_params=pltpu.CompilerParams(use_tc_tiling_on_sc=False),
        scratch_types=[
            pltpu.VMEM((n_chunk_rows, CHUNK), _I32),
            pltpu.VMEM((n_chunk_rows, CHUNK), _F32),
            pltpu.VMEM((seg,), _F32),
            pltpu.VMEM_SHARED((n_pad,), _F32),
        ],
    )
    def deg_kernel(dst2d, ew2d, dis_out, dst_all, ew_all, buf, acc):
        c = lax.axis_index("c")
        s = lax.axis_index("s")

        @pl.when(c == 0)
        def _():
            z = jnp.zeros((LANES,), _F32)

            def zb(i, _):
                buf[pl.ds(i * LANES, LANES)] = z
                return 0

            lax.fori_loop(0, seg // LANES, zb, 0)
            pltpu.sync_copy(buf, acc.at[pl.ds(s * seg, seg)])
            plsc.subcore_barrier()

            pltpu.sync_copy(dst2d.at[pl.ds(s * n_chunk_rows, n_chunk_rows)], dst_all)
            pltpu.sync_copy(ew2d.at[pl.ds(s * n_chunk_rows, n_chunk_rows)], ew_all)

            def chunk(g, _):
                pltpu.sync_copy(ew_all.at[g], acc.at[dst_all.at[g]], add=True)
                return 0

            lax.fori_loop(0, n_chunk_rows, chunk, 0)
            plsc.subcore_barrier()

            pltpu.sync_copy(acc.at[pl.ds(s * seg, seg)], buf)

            def newton(i, _):
                sl = pl.ds(i * LANES, LANES)
                d = buf[sl] + 1.0
                pos = d > 0.0
                bits = lax.bitcast_convert_type(d, _I32)
                y = lax.bitcast_convert_type(
                    jnp.int32(0x5F3759DF) - lax.shift_right_arithmetic(bits, 1), _F32
                )
                half = d * 0.5
                for _it in range(4):
                    y = y * (1.5 - half * y * y)
                buf[sl] = jnp.where(pos, y, 0.0)
                return 0

            lax.fori_loop(0, seg // LANES, newton, 0)
            pltpu.sync_copy(buf, dis_out.at[pl.ds(s * seg, seg)])

    return deg_kernel


def _make_sc_agg(n_pad, d, n_chunk_rows):
    seg = n_pad // NS

    @functools.partial(
        pl.kernel,
        mesh=plsc.VectorSubcoreMesh(core_axis_name="c", subcore_axis_name="s"),
        out_type=pltpu.HBM((NC, n_pad, d), _F32),
        compiler_params=pltpu.CompilerParams(use_tc_tiling_on_sc=False),
        scratch_types=[
            pltpu.VMEM((n_chunk_rows, CHUNK), _I32),
            pltpu.VMEM((n_chunk_rows, CHUNK), _I32),
            pltpu.VMEM((n_chunk_rows, CHUNK), _F32),
            pltpu.VMEM((CHUNK, d), _F32),
            pltpu.VMEM((CHUNK, d), _F32),
            pltpu.VMEM((128, d), _F32),
            pltpu.VMEM_SHARED((n_pad, d), _F32),
            pltpu.SemaphoreType.DMA,
            pltpu.SemaphoreType.DMA,
            pltpu.SemaphoreType.DMA,
            pltpu.SemaphoreType.DMA,
        ],
    )
    def agg_kernel(src2d, dst2d, ew2d, hs, out, src_all, dst_all, ew_all,
                   rows_a, rows_b, zb, acc, sem_a, sem_b, ssem_a, ssem_b):
        c = lax.axis_index("c")
        s = lax.axis_index("s")
        wid = c * NS + s

        _zero_rows(zb, d)
        for i in range(seg // 128):
            pltpu.sync_copy(zb, acc.at[pl.ds(s * seg + i * 128, 128)])
        plsc.subcore_barrier()

        pltpu.sync_copy(src2d.at[pl.ds(wid * n_chunk_rows, n_chunk_rows)], src_all)
        pltpu.sync_copy(dst2d.at[pl.ds(wid * n_chunk_rows, n_chunk_rows)], dst_all)
        pltpu.sync_copy(ew2d.at[pl.ds(wid * n_chunk_rows, n_chunk_rows)], ew_all)

        def gather(g, buf, sem):
            pltpu.async_copy(hs.at[src_all.at[g]], buf, sem)

        def scale(g, buf, gsem):
            pltpu.make_async_copy(hs.at[src_all.at[g]], buf, gsem).wait()

            def body(jb, _):
                ewv = ew_all[g, pl.ds(jb * LANES, LANES)]
                for jj in range(LANES):
                    cv = jnp.full((LANES,), ewv[jj], _F32)
                    j = jb * LANES + jj
                    for k in range(d // LANES):
                        sl = pl.ds(k * LANES, LANES)
                        buf[j, sl] = buf[j, sl] * cv
                return 0

            lax.fori_loop(0, CHUNK // LANES, body, 0)

        def scatter(g, buf, ssem):
            pltpu.async_copy(buf, acc.at[dst_all.at[g]], ssem, add=True)

        def scatter_wait(g, buf, ssem):
            pltpu.make_async_copy(buf, acc.at[dst_all.at[g]], ssem).wait()

        gather(0, rows_a, sem_a)
        gather(1, rows_b, sem_b)

        def pair(g2, _):
            g = g2 * 2
            scale(g, rows_a, sem_a)
            scatter(g, rows_a, ssem_a)
            scale(g + 1, rows_b, sem_b)
            scatter(g + 1, rows_b, ssem_b)
            scatter_wait(g, rows_a, ssem_a)
            gather(g + 2, rows_a, sem_a)
            scatter_wait(g + 1, rows_b, ssem_b)
            gather(g + 3, rows_b, sem_b)
            return 0

        lax.fori_loop(0, n_chunk_rows // 2 - 1, pair, 0)
        g_last = n_chunk_rows - 2
        scale(g_last, rows_a, sem_a)
        pltpu.sync_copy(rows_a, acc.at[dst_all.at[g_last]], add=True)
        scale(g_last + 1, rows_b, sem_b)
        pltpu.sync_copy(rows_b, acc.at[dst_all.at[g_last + 1]], add=True)
        plsc.subcore_barrier()

        for i in range(seg // 128):
            r0 = s * seg + i * 128
            pltpu.sync_copy(acc.at[pl.ds(r0, 128)], out.at[c, pl.ds(r0, 128)])

    return agg_kernel


def _tc_scale(x, w, dis_col, n_pad):
    n = x.shape[0]
    h = w.shape[1]

    def body(x_ref, w_ref, d_ref, o_ref):
        hm = jnp.dot(x_ref[...], w_ref[...], preferred_element_type=_F32)
        o_ref[pl.ds(0, n), :] = hm * d_ref[...]
        o_ref[pl.ds(n, n_pad - n), :] = jnp.zeros((n_pad - n, h), _F32)

    return pl.pallas_call(
        body, out_shape=jax.ShapeDtypeStruct((n_pad, h), _F32)
    )(x, w, dis_col)


def _tc_mid(p0, p1, hs1, dis_col, b1, w2p, n_pad):
    n, _h = hs1.shape
    d2 = w2p.shape[1]

    def body(p0_ref, p1_ref, hs_ref, d_ref, b_ref, w_ref, o_ref):
        t = (p0_ref[...] + p1_ref[...] + hs_ref[...]) * d_ref[...] + b_ref[...]
        r = jnp.maximum(t, 0.0)
        o_ref[pl.ds(0, n), :] = (
            jnp.dot(r, w_ref[...], preferred_element_type=_F32) * d_ref[...]
        )
        o_ref[pl.ds(n, n_pad - n), :] = jnp.zeros((n_pad - n, d2), _F32)

    return pl.pallas_call(
        body, out_shape=jax.ShapeDtypeStruct((n_pad, d2), _F32)
    )(p0, p1, hs1, dis_col, b1, w2p)


def _tc_final(q0, q1, hs2p, dis_col, b2):
    n = hs2p.shape[0]
    c_out = b2.shape[0]

    def body(q0_ref, q1_ref, hs_ref, d_ref, b_ref, o_ref):
        t = (q0_ref[...] + q1_ref[...] + hs_ref[...]) * d_ref[...]
        o_ref[...] = t[:, :c_out] + b_ref[...]

    return pl.pallas_call(
        body, out_shape=jax.ShapeDtypeStruct((n, c_out), _F32)
    )(q0, q1, hs2p, dis_col, b2)


def kernel(x, edge_index, edge_attr, W1, b1, W2, b2):
    n, _dim = x.shape
    e = edge_attr.shape[0]
    n_pad = -(-n // (NS * ROWS_PER_TILE)) * (NS * ROWS_PER_TILE)

    align = NC * NS * CHUNK * 8
    e_pad = -(-e // align) * align
    pad = e_pad - e
    src = jnp.pad(edge_index[0], (0, pad)).reshape(-1, CHUNK)
    dst = jnp.pad(edge_index[1], (0, pad)).reshape(-1, CHUNK)
    ew = jnp.pad(edge_attr, (0, pad)).reshape(-1, CHUNK)

    deg_rows = e_pad // (NS * CHUNK)
    agg_rows = e_pad // (NC * NS * CHUNK)

    dis_pad = _make_sc_deg(n_pad, deg_rows)(dst, ew)
    dis_col = dis_pad[:n].reshape(n, 1)

    hs1 = _tc_scale(x, W1, dis_col, n_pad)
    p = _make_sc_agg(n_pad, W1.shape[1], agg_rows)(src, dst, ew, hs1)

    d2p = -(-W2.shape[1] // LANES) * LANES
    w2p = jnp.pad(W2, ((0, 0), (0, d2p - W2.shape[1])))
    hs2p = _tc_mid(p[0, :n], p[1, :n], hs1[:n], dis_col, b1, w2p, n_pad)

    q = _make_sc_agg(n_pad, d2p, agg_rows)(src, dst, ew, hs2p)
    return _tc_final(q[0, :n], q[1, :n], hs2p[:n], dis_col, b2)


# speedup vs baseline: 1.2039x; 1.2039x over previous
import functools

import jax
import jax.numpy as jnp
from jax import lax
from jax.experimental import pallas as pl
from jax.experimental.pallas import tpu as pltpu
from jax.experimental.pallas import tpu_sc as plsc

NC = 2
NS = 16
LANES = 16
CHUNK = 128
ROWS_PER_TILE = 640

_F32 = jnp.float32
_I32 = jnp.int32


def _zero_rows(zb, d):
    z = jnp.zeros((LANES,), _F32)

    def body(i, _):
        for k in range(d // LANES):
            zb[i, pl.ds(k * LANES, LANES)] = z
        return 0

    lax.fori_loop(0, 128, body, 0)


def _make_sc_deg(n_pad, n_chunk_rows):
    seg = n_pad // NS

    @functools.partial(
        pl.kernel,
        mesh=plsc.VectorSubcoreMesh(core_axis_name="c", subcore_axis_name="s"),
        out_type=jax.ShapeDtypeStruct((n_pad,), _F32),
        compiler_params=pltpu.CompilerParams(use_tc_tiling_on_sc=False),
        scratch_types=[
            pltpu.VMEM((n_chunk_rows, CHUNK), _I32),
            pltpu.VMEM((n_chunk_rows, CHUNK), _F32),
            pltpu.VMEM((seg,), _F32),
            pltpu.VMEM_SHARED((n_pad,), _F32),
        ],
    )
    def deg_kernel(dst2d, ew2d, dis_out, dst_all, ew_all, buf, acc):
        c = lax.axis_index("c")
        s = lax.axis_index("s")

        @pl.when(c == 0)
        def _():
            z = jnp.zeros((LANES,), _F32)

            def zb(i, _):
                buf[pl.ds(i * LANES, LANES)] = z
                return 0

            lax.fori_loop(0, seg // LANES, zb, 0)
            pltpu.sync_copy(buf, acc.at[pl.ds(s * seg, seg)])
            plsc.subcore_barrier()

            pltpu.sync_copy(dst2d.at[pl.ds(s * n_chunk_rows, n_chunk_rows)], dst_all)
            pltpu.sync_copy(ew2d.at[pl.ds(s * n_chunk_rows, n_chunk_rows)], ew_all)

            def chunk(g, _):
                pltpu.sync_copy(ew_all.at[g], acc.at[dst_all.at[g]], add=True)
                return 0

            lax.fori_loop(0, n_chunk_rows, chunk, 0)
            plsc.subcore_barrier()

            pltpu.sync_copy(acc.at[pl.ds(s * seg, seg)], buf)

            def newton(i, _):
                sl = pl.ds(i * LANES, LANES)
                d = buf[sl] + 1.0
                pos = d > 0.0
                bits = lax.bitcast_convert_type(d, _I32)
                y = lax.bitcast_convert_type(
                    jnp.int32(0x5F3759DF) - lax.shift_right_arithmetic(bits, 1), _F32
                )
                half = d * 0.5
                for _it in range(4):
                    y = y * (1.5 - half * y * y)
                buf[sl] = jnp.where(pos, y, 0.0)
                return 0

            lax.fori_loop(0, seg // LANES, newton, 0)
            pltpu.sync_copy(buf, dis_out.at[pl.ds(s * seg, seg)])

    return deg_kernel


def _make_sc_agg(n_pad, d, n_chunk_rows):
    seg = n_pad // NS

    @functools.partial(
        pl.kernel,
        mesh=plsc.VectorSubcoreMesh(core_axis_name="c", subcore_axis_name="s"),
        out_type=pltpu.HBM((NC, n_pad, d), _F32),
        compiler_params=pltpu.CompilerParams(
            use_tc_tiling_on_sc=False, needs_layout_passes=False
        ),
        scratch_types=[
            pltpu.VMEM((n_chunk_rows, CHUNK), _I32),
            pltpu.VMEM((n_chunk_rows, CHUNK), _I32),
            pltpu.VMEM((n_chunk_rows, CHUNK), _F32),
            pltpu.VMEM((CHUNK, d), _F32),
            pltpu.VMEM((CHUNK, d), _F32),
            pltpu.VMEM((128, d), _F32),
            pltpu.VMEM_SHARED((n_pad, d), _F32),
            pltpu.SemaphoreType.DMA,
            pltpu.SemaphoreType.DMA,
        ],
    )
    def agg_kernel(src2d, dst2d, ew2d, hs, out, src_all, dst_all, ew_all,
                   rows_a, rows_b, zb, acc, sem_a, sem_b):
        c = lax.axis_index("c")
        s = lax.axis_index("s")
        wid = c * NS + s

        _zero_rows(zb, d)
        for i in range(seg // 128):
            pltpu.sync_copy(zb, acc.at[pl.ds(s * seg + i * 128, 128)])
        plsc.subcore_barrier()

        pltpu.sync_copy(src2d.at[pl.ds(wid * n_chunk_rows, n_chunk_rows)], src_all)
        pltpu.sync_copy(dst2d.at[pl.ds(wid * n_chunk_rows, n_chunk_rows)], dst_all)
        pltpu.sync_copy(ew2d.at[pl.ds(wid * n_chunk_rows, n_chunk_rows)], ew_all)

        def gather(g, buf, sem):
            pltpu.async_copy(hs.at[src_all.at[g]], buf, sem)

        def drain(g, buf, sem):
            pltpu.make_async_copy(hs.at[src_all.at[g]], buf, sem).wait()
            gv = jnp.full((LANES,), g, _I32)

            def body(j, _):
                cv = plsc.load_gather(ew_all, [gv, jnp.full((LANES,), j, _I32)])
                for k in range(d // LANES):
                    sl = pl.ds(k * LANES, LANES)
                    buf[j, sl] = buf[j, sl] * cv
                return 0

            lax.fori_loop(0, CHUNK, body, 0)
            pltpu.sync_copy(buf, acc.at[dst_all.at[g]], add=True)

        gather(0, rows_a, sem_a)

        def pair(g2, _):
            g = g2 * 2
            gather(g + 1, rows_b, sem_b)
            drain(g, rows_a, sem_a)
            gather(g + 2, rows_a, sem_a)
            drain(g + 1, rows_b, sem_b)
            return 0

        lax.fori_loop(0, n_chunk_rows // 2 - 1, pair, 0)
        g_last = n_chunk_rows - 2
        gather(g_last + 1, rows_b, sem_b)
        drain(g_last, rows_a, sem_a)
        drain(g_last + 1, rows_b, sem_b)
        plsc.subcore_barrier()

        for i in range(seg // 128):
            r0 = s * seg + i * 128
            pltpu.sync_copy(acc.at[pl.ds(r0, 128)], out.at[c, pl.ds(r0, 128)])

    return agg_kernel


def _tc_scale(x, w, dis_col, n_pad):
    n = x.shape[0]
    h = w.shape[1]

    def body(x_ref, w_ref, d_ref, o_ref):
        hm = jnp.dot(x_ref[...], w_ref[...], preferred_element_type=_F32)
        o_ref[pl.ds(0, n), :] = hm * d_ref[...]
        o_ref[pl.ds(n, n_pad - n), :] = jnp.zeros((n_pad - n, h), _F32)

    return pl.pallas_call(
        body, out_shape=jax.ShapeDtypeStruct((n_pad, h), _F32)
    )(x, w, dis_col)


def _tc_mid(p0, p1, hs1, dis_col, b1, w2p, n_pad):
    n, _h = hs1.shape
    d2 = w2p.shape[1]

    def body(p0_ref, p1_ref, hs_ref, d_ref, b_ref, w_ref, o_ref):
        t = (p0_ref[...] + p1_ref[...] + hs_ref[...]) * d_ref[...] + b_ref[...]
        r = jnp.maximum(t, 0.0)
        o_ref[pl.ds(0, n), :] = (
            jnp.dot(r, w_ref[...], preferred_element_type=_F32) * d_ref[...]
        )
        o_ref[pl.ds(n, n_pad - n), :] = jnp.zeros((n_pad - n, d2), _F32)

    return pl.pallas_call(
        body, out_shape=jax.ShapeDtypeStruct((n_pad, d2), _F32)
    )(p0, p1, hs1, dis_col, b1, w2p)


def _tc_final(q0, q1, hs2p, dis_col, b2):
    n = hs2p.shape[0]
    c_out = b2.shape[0]

    def body(q0_ref, q1_ref, hs_ref, d_ref, b_ref, o_ref):
        t = (q0_ref[...] + q1_ref[...] + hs_ref[...]) * d_ref[...]
        o_ref[...] = t[:, :c_out] + b_ref[...]

    return pl.pallas_call(
        body, out_shape=jax.ShapeDtypeStruct((n, c_out), _F32)
    )(q0, q1, hs2p, dis_col, b2)


def kernel(x, edge_index, edge_attr, W1, b1, W2, b2):
    n, _dim = x.shape
    e = edge_attr.shape[0]
    n_pad = -(-n // (NS * ROWS_PER_TILE)) * (NS * ROWS_PER_TILE)

    align = NC * NS * CHUNK * 8
    e_pad = -(-e // align) * align
    pad = e_pad - e
    src = jnp.pad(edge_index[0], (0, pad)).reshape(-1, CHUNK)
    dst = jnp.pad(edge_index[1], (0, pad)).reshape(-1, CHUNK)
    ew = jnp.pad(edge_attr, (0, pad)).reshape(-1, CHUNK)

    deg_rows = e_pad // (NS * CHUNK)
    agg_rows = e_pad // (NC * NS * CHUNK)

    dis_pad = _make_sc_deg(n_pad, deg_rows)(dst, ew)
    dis_col = dis_pad[:n].reshape(n, 1)

    hs1 = _tc_scale(x, W1, dis_col, n_pad)
    p = _make_sc_agg(n_pad, W1.shape[1], agg_rows)(src, dst, ew, hs1)

    d2p = -(-W2.shape[1] // LANES) * LANES
    w2p = jnp.pad(W2, ((0, 0), (0, d2p - W2.shape[1])))
    hs2p = _tc_mid(p[0, :n], p[1, :n], hs1[:n], dis_col, b1, w2p, n_pad)

    q = _make_sc_agg(n_pad, d2p, agg_rows)(src, dst, ew, hs2p)
    return _tc_final(q[0, :n], q[1, :n], hs2p[:n], dis_col, b2)


# speedup vs baseline: 1.2352x; 1.0260x over previous
import functools

import jax
import jax.numpy as jnp
from jax import lax
from jax.experimental import pallas as pl
from jax.experimental.pallas import tpu as pltpu
from jax.experimental.pallas import tpu_sc as plsc

NC = 2
NS = 16
LANES = 16
CHUNK = 128
ROWS_PER_TILE = 640

_F32 = jnp.float32
_I32 = jnp.int32


def _zero_rows(zb, d):
    z = jnp.zeros((LANES,), _F32)

    def body(i, _):
        for k in range(d // LANES):
            zb[i, pl.ds(k * LANES, LANES)] = z
        return 0

    lax.fori_loop(0, 128, body, 0)


def _make_sc_deg(n_pad, n_chunk_rows):
    seg = n_pad // NS

    @functools.partial(
        pl.kernel,
        mesh=plsc.VectorSubcoreMesh(core_axis_name="c", subcore_axis_name="s"),
        out_type=jax.ShapeDtypeStruct((n_pad,), _F32),
        compiler_params=pltpu.CompilerParams(use_tc_tiling_on_sc=False),
        scratch_types=[
            pltpu.VMEM((n_chunk_rows, CHUNK), _I32),
            pltpu.VMEM((n_chunk_rows, CHUNK), _F32),
            pltpu.VMEM((seg,), _F32),
            pltpu.VMEM_SHARED((n_pad,), _F32),
        ],
    )
    def deg_kernel(dst2d, ew2d, dis_out, dst_all, ew_all, buf, acc):
        c = lax.axis_index("c")
        s = lax.axis_index("s")

        @pl.when(c == 0)
        def _():
            z = jnp.zeros((LANES,), _F32)

            def zb(i, _):
                buf[pl.ds(i * LANES, LANES)] = z
                return 0

            lax.fori_loop(0, seg // LANES, zb, 0)
            pltpu.sync_copy(buf, acc.at[pl.ds(s * seg, seg)])
            plsc.subcore_barrier()

            pltpu.sync_copy(dst2d.at[pl.ds(s * n_chunk_rows, n_chunk_rows)], dst_all)
            pltpu.sync_copy(ew2d.at[pl.ds(s * n_chunk_rows, n_chunk_rows)], ew_all)

            def chunk(g, _):
                pltpu.sync_copy(ew_all.at[g], acc.at[dst_all.at[g]], add=True)
                return 0

            lax.fori_loop(0, n_chunk_rows, chunk, 0)
            plsc.subcore_barrier()

            pltpu.sync_copy(acc.at[pl.ds(s * seg, seg)], buf)

            def newton(i, _):
                sl = pl.ds(i * LANES, LANES)
                d = buf[sl] + 1.0
                pos = d > 0.0
                bits = lax.bitcast_convert_type(d, _I32)
                y = lax.bitcast_convert_type(
                    jnp.int32(0x5F3759DF) - lax.shift_right_arithmetic(bits, 1), _F32
                )
                half = d * 0.5
                for _it in range(4):
                    y = y * (1.5 - half * y * y)
                buf[sl] = jnp.where(pos, y, 0.0)
                return 0

            lax.fori_loop(0, seg // LANES, newton, 0)
            pltpu.sync_copy(buf, dis_out.at[pl.ds(s * seg, seg)])

    return deg_kernel


def _make_sc_agg(n_pad, d, n_chunk_rows):
    seg = n_pad // NS

    @functools.partial(
        pl.kernel,
        mesh=plsc.VectorSubcoreMesh(core_axis_name="c", subcore_axis_name="s"),
        out_type=pltpu.HBM((NC, n_pad, d), _F32),
        compiler_params=pltpu.CompilerParams(
            use_tc_tiling_on_sc=False, needs_layout_passes=False
        ),
        scratch_types=[
            pltpu.VMEM((n_chunk_rows, CHUNK), _I32),
            pltpu.VMEM((n_chunk_rows, CHUNK), _I32),
            pltpu.VMEM((n_chunk_rows, CHUNK), _F32),
            [pltpu.VMEM((CHUNK, d), _F32) for _ in range(4)],
            pltpu.VMEM((128, d), _F32),
            pltpu.VMEM_SHARED((n_pad, d), _F32),
            [pltpu.SemaphoreType.DMA for _ in range(4)],
        ],
    )
    def agg_kernel(src2d, dst2d, ew2d, hs, out, src_all, dst_all, ew_all,
                   bufs, zb, acc, sems):
        c = lax.axis_index("c")
        s = lax.axis_index("s")
        wid = c * NS + s

        _zero_rows(zb, d)
        for i in range(seg // 128):
            pltpu.sync_copy(zb, acc.at[pl.ds(s * seg + i * 128, 128)])
        plsc.subcore_barrier()

        pltpu.sync_copy(src2d.at[pl.ds(wid * n_chunk_rows, n_chunk_rows)], src_all)
        pltpu.sync_copy(dst2d.at[pl.ds(wid * n_chunk_rows, n_chunk_rows)], dst_all)
        pltpu.sync_copy(ew2d.at[pl.ds(wid * n_chunk_rows, n_chunk_rows)], ew_all)

        n_rows = n_chunk_rows

        def gather(g, buf, sem):
            pltpu.async_copy(hs.at[src_all.at[g]], buf, sem)

        def gather_wait(g, buf, sem):
            pltpu.make_async_copy(hs.at[src_all.at[g]], buf, sem).wait()

        def scatter(g, buf, sem):
            pltpu.async_copy(buf, acc.at[dst_all.at[g]], sem, add=True)

        def scatter_wait(g, buf, sem):
            pltpu.make_async_copy(buf, acc.at[dst_all.at[g]], sem).wait()

        def scale(g, buf):
            gv = jnp.full((LANES,), g, _I32)

            def body(j, _):
                cv = plsc.load_gather(ew_all, [gv, jnp.full((LANES,), j, _I32)])
                for k in range(d // LANES):
                    sl = pl.ds(k * LANES, LANES)
                    buf[j, sl] = buf[j, sl] * cv
                return 0

            lax.fori_loop(0, CHUNK, body, 0)

        gather(0, bufs[0], sems[0])
        gather(1, bufs[1], sems[1])

        def quad(g4, _):
            for j in range(4):
                g = g4 * 4 + j
                buf, sem = bufs[j], sems[j]
                nbuf, nsem = bufs[(j + 2) % 4], sems[(j + 2) % 4]
                gather_wait(g, buf, sem)
                scale(g, buf)
                scatter(g, buf, sem)

                @pl.when(jnp.logical_and(g + 2 < n_rows, g >= 2))
                def _():
                    scatter_wait(g - 2, nbuf, nsem)

                @pl.when(g + 2 < n_rows)
                def _():
                    gather(g + 2, nbuf, nsem)
            return 0

        lax.fori_loop(0, n_rows // 4, quad, 0)
        for j in range(4):
            scatter_wait(n_rows - 4 + j, bufs[j], sems[j])
        plsc.subcore_barrier()

        for i in range(seg // 128):
            r0 = s * seg + i * 128
            pltpu.sync_copy(acc.at[pl.ds(r0, 128)], out.at[c, pl.ds(r0, 128)])

    return agg_kernel


def _tc_scale(x, w, dis_col, n_pad):
    n = x.shape[0]
    h = w.shape[1]

    def body(x_ref, w_ref, d_ref, o_ref):
        hm = jnp.dot(x_ref[...], w_ref[...], preferred_element_type=_F32)
        o_ref[pl.ds(0, n), :] = hm * d_ref[...]
        o_ref[pl.ds(n, n_pad - n), :] = jnp.zeros((n_pad - n, h), _F32)

    return pl.pallas_call(
        body, out_shape=jax.ShapeDtypeStruct((n_pad, h), _F32)
    )(x, w, dis_col)


def _tc_mid(p0, p1, hs1, dis_col, b1, w2p, n_pad):
    n, _h = hs1.shape
    d2 = w2p.shape[1]

    def body(p0_ref, p1_ref, hs_ref, d_ref, b_ref, w_ref, o_ref):
        t = (p0_ref[...] + p1_ref[...] + hs_ref[...]) * d_ref[...] + b_ref[...]
        r = jnp.maximum(t, 0.0)
        o_ref[pl.ds(0, n), :] = (
            jnp.dot(r, w_ref[...], preferred_element_type=_F32) * d_ref[...]
        )
        o_ref[pl.ds(n, n_pad - n), :] = jnp.zeros((n_pad - n, d2), _F32)

    return pl.pallas_call(
        body, out_shape=jax.ShapeDtypeStruct((n_pad, d2), _F32)
    )(p0, p1, hs1, dis_col, b1, w2p)


def _tc_final(q0, q1, hs2p, dis_col, b2):
    n = hs2p.shape[0]
    c_out = b2.shape[0]

    def body(q0_ref, q1_ref, hs_ref, d_ref, b_ref, o_ref):
        t = (q0_ref[...] + q1_ref[...] + hs_ref[...]) * d_ref[...]
        o_ref[...] = t[:, :c_out] + b_ref[...]

    return pl.pallas_call(
        body, out_shape=jax.ShapeDtypeStruct((n, c_out), _F32)
    )(q0, q1, hs2p, dis_col, b2)


def kernel(x, edge_index, edge_attr, W1, b1, W2, b2):
    n, _dim = x.shape
    e = edge_attr.shape[0]
    n_pad = -(-n // (NS * ROWS_PER_TILE)) * (NS * ROWS_PER_TILE)

    align = NC * NS * CHUNK * 8
    e_pad = -(-e // align) * align
    pad = e_pad - e
    src = jnp.pad(edge_index[0], (0, pad)).reshape(-1, CHUNK)
    dst = jnp.pad(edge_index[1], (0, pad)).reshape(-1, CHUNK)
    ew = jnp.pad(edge_attr, (0, pad)).reshape(-1, CHUNK)

    deg_rows = e_pad // (NS * CHUNK)
    agg_rows = e_pad // (NC * NS * CHUNK)

    dis_pad = _make_sc_deg(n_pad, deg_rows)(dst, ew)
    dis_col = dis_pad[:n].reshape(n, 1)

    hs1 = _tc_scale(x, W1, dis_col, n_pad)
    p = _make_sc_agg(n_pad, W1.shape[1], agg_rows)(src, dst, ew, hs1)

    d2p = -(-W2.shape[1] // LANES) * LANES
    w2p = jnp.pad(W2, ((0, 0), (0, d2p - W2.shape[1])))
    hs2p = _tc_mid(p[0, :n], p[1, :n], hs1[:n], dis_col, b1, w2p, n_pad)

    q = _make_sc_agg(n_pad, d2p, agg_rows)(src, dst, ew, hs2p)
    return _tc_final(q[0, :n], q[1, :n], hs2p[:n], dis_col, b2)


# speedup vs baseline: 1.2489x; 1.0111x over previous
import functools

import jax
import jax.numpy as jnp
from jax import lax
from jax.experimental import pallas as pl
from jax.experimental.pallas import tpu as pltpu
from jax.experimental.pallas import tpu_sc as plsc

NC = 2
NS = 16
LANES = 16
CHUNK = 128
ROWS_PER_TILE = 640
NBUF = 5
AHEAD = 4

_F32 = jnp.float32
_I32 = jnp.int32


def _zero_rows(zb, d):
    z = jnp.zeros((LANES,), _F32)

    def body(i, _):
        for k in range(d // LANES):
            zb[i, pl.ds(k * LANES, LANES)] = z
        return 0

    lax.fori_loop(0, 128, body, 0)


def _make_sc_deg(n_pad, n_chunk_rows):
    seg = n_pad // NS

    @functools.partial(
        pl.kernel,
        mesh=plsc.VectorSubcoreMesh(core_axis_name="c", subcore_axis_name="s"),
        out_type=jax.ShapeDtypeStruct((n_pad,), _F32),
        compiler_params=pltpu.CompilerParams(use_tc_tiling_on_sc=False),
        scratch_types=[
            pltpu.VMEM((n_chunk_rows, CHUNK), _I32),
            pltpu.VMEM((n_chunk_rows, CHUNK), _F32),
            pltpu.VMEM((seg,), _F32),
            pltpu.VMEM_SHARED((n_pad,), _F32),
        ],
    )
    def deg_kernel(dst2d, ew2d, dis_out, dst_all, ew_all, buf, acc):
        c = lax.axis_index("c")
        s = lax.axis_index("s")

        @pl.when(c == 0)
        def _():
            z = jnp.zeros((LANES,), _F32)

            def zb(i, _):
                buf[pl.ds(i * LANES, LANES)] = z
                return 0

            lax.fori_loop(0, seg // LANES, zb, 0)
            pltpu.sync_copy(buf, acc.at[pl.ds(s * seg, seg)])
            plsc.subcore_barrier()

            pltpu.sync_copy(dst2d.at[pl.ds(s * n_chunk_rows, n_chunk_rows)], dst_all)
            pltpu.sync_copy(ew2d.at[pl.ds(s * n_chunk_rows, n_chunk_rows)], ew_all)

            def chunk(g, _):
                pltpu.sync_copy(ew_all.at[g], acc.at[dst_all.at[g]], add=True)
                return 0

            lax.fori_loop(0, n_chunk_rows, chunk, 0)
            plsc.subcore_barrier()

            pltpu.sync_copy(acc.at[pl.ds(s * seg, seg)], buf)

            def newton(i, _):
                sl = pl.ds(i * LANES, LANES)
                d = buf[sl] + 1.0
                pos = d > 0.0
                bits = lax.bitcast_convert_type(d, _I32)
                y = lax.bitcast_convert_type(
                    jnp.int32(0x5F3759DF) - lax.shift_right_arithmetic(bits, 1), _F32
                )
                half = d * 0.5
                for _it in range(4):
                    y = y * (1.5 - half * y * y)
                buf[sl] = jnp.where(pos, y, 0.0)
                return 0

            lax.fori_loop(0, seg // LANES, newton, 0)
            pltpu.sync_copy(buf, dis_out.at[pl.ds(s * seg, seg)])

    return deg_kernel


def _make_sc_agg(n_pad, d, n_chunk_rows):
    seg = n_pad // NS

    @functools.partial(
        pl.kernel,
        mesh=plsc.VectorSubcoreMesh(core_axis_name="c", subcore_axis_name="s"),
        out_type=pltpu.HBM((NC, n_pad, d), _F32),
        compiler_params=pltpu.CompilerParams(
            use_tc_tiling_on_sc=False, needs_layout_passes=False
        ),
        scratch_types=[
            pltpu.VMEM((n_chunk_rows, CHUNK), _I32),
            pltpu.VMEM((n_chunk_rows, CHUNK), _I32),
            pltpu.VMEM((n_chunk_rows, CHUNK), _F32),
            [pltpu.VMEM((CHUNK, d), _F32) for _ in range(NBUF)],
            pltpu.VMEM((128, d), _F32),
            pltpu.VMEM_SHARED((n_pad, d), _F32),
            [pltpu.SemaphoreType.DMA for _ in range(NBUF)],
        ],
    )
    def agg_kernel(src2d, dst2d, ew2d, hs, out, src_all, dst_all, ew_all,
                   bufs, zb, acc, sems):
        c = lax.axis_index("c")
        s = lax.axis_index("s")
        wid = c * NS + s

        _zero_rows(zb, d)
        for i in range(seg // 128):
            pltpu.sync_copy(zb, acc.at[pl.ds(s * seg + i * 128, 128)])
        plsc.subcore_barrier()

        pltpu.sync_copy(src2d.at[pl.ds(wid * n_chunk_rows, n_chunk_rows)], src_all)
        pltpu.sync_copy(dst2d.at[pl.ds(wid * n_chunk_rows, n_chunk_rows)], dst_all)
        pltpu.sync_copy(ew2d.at[pl.ds(wid * n_chunk_rows, n_chunk_rows)], ew_all)

        n_rows = n_chunk_rows

        def gather(g, buf, sem):
            pltpu.async_copy(hs.at[src_all.at[g]], buf, sem)

        def gather_wait(g, buf, sem):
            pltpu.make_async_copy(hs.at[src_all.at[g]], buf, sem).wait()

        def scatter(g, buf, sem):
            pltpu.async_copy(buf, acc.at[dst_all.at[g]], sem, add=True)

        def scatter_wait(g, buf, sem):
            pltpu.make_async_copy(buf, acc.at[dst_all.at[g]], sem).wait()

        def scale(g, buf):
            gv = jnp.full((LANES,), g, _I32)

            def body(j, _):
                cv = plsc.load_gather(ew_all, [gv, jnp.full((LANES,), j, _I32)])
                for k in range(d // LANES):
                    sl = pl.ds(k * LANES, LANES)
                    buf[j, sl] = buf[j, sl] * cv
                return 0

            lax.fori_loop(0, CHUNK, body, 0)

        for j in range(AHEAD):
            gather(j, bufs[j], sems[j])

        def rotation(gq, _):
            for j in range(NBUF):
                g = gq * NBUF + j
                buf, sem = bufs[j], sems[j]
                jn = (j + AHEAD) % NBUF
                nbuf, nsem = bufs[jn], sems[jn]
                gather_wait(g, buf, sem)
                scale(g, buf)
                scatter(g, buf, sem)

                @pl.when(
                    jnp.logical_and(g + AHEAD < n_rows, g + AHEAD >= NBUF)
                )
                def _():
                    scatter_wait(g + AHEAD - NBUF, nbuf, nsem)

                @pl.when(g + AHEAD < n_rows)
                def _():
                    gather(g + AHEAD, nbuf, nsem)
            return 0

        lax.fori_loop(0, n_rows // NBUF, rotation, 0)
        for j in range(NBUF):
            scatter_wait(n_rows - NBUF + j, bufs[j], sems[j])
        plsc.subcore_barrier()

        for i in range(seg // 128):
            r0 = s * seg + i * 128
            pltpu.sync_copy(acc.at[pl.ds(r0, 128)], out.at[c, pl.ds(r0, 128)])

    return agg_kernel


def _tc_scale(x, w, dis_col, n_pad):
    n = x.shape[0]
    h = w.shape[1]

    def body(x_ref, w_ref, d_ref, o_ref):
        hm = jnp.dot(x_ref[...], w_ref[...], preferred_element_type=_F32)
        o_ref[pl.ds(0, n), :] = hm * d_ref[...]
        o_ref[pl.ds(n, n_pad - n), :] = jnp.zeros((n_pad - n, h), _F32)

    return pl.pallas_call(
        body, out_shape=jax.ShapeDtypeStruct((n_pad, h), _F32)
    )(x, w, dis_col)


def _tc_mid(p0, p1, hs1, dis_col, b1, w2p, n_pad):
    n, _h = hs1.shape
    d2 = w2p.shape[1]

    def body(p0_ref, p1_ref, hs_ref, d_ref, b_ref, w_ref, o_ref):
        t = (p0_ref[...] + p1_ref[...] + hs_ref[...]) * d_ref[...] + b_ref[...]
        r = jnp.maximum(t, 0.0)
        o_ref[pl.ds(0, n), :] = (
            jnp.dot(r, w_ref[...], preferred_element_type=_F32) * d_ref[...]
        )
        o_ref[pl.ds(n, n_pad - n), :] = jnp.zeros((n_pad - n, d2), _F32)

    return pl.pallas_call(
        body, out_shape=jax.ShapeDtypeStruct((n_pad, d2), _F32)
    )(p0, p1, hs1, dis_col, b1, w2p)


def _tc_final(q0, q1, hs2p, dis_col, b2):
    n = hs2p.shape[0]
    c_out = b2.shape[0]

    def body(q0_ref, q1_ref, hs_ref, d_ref, b_ref, o_ref):
        t = (q0_ref[...] + q1_ref[...] + hs_ref[...]) * d_ref[...]
        o_ref[...] = t[:, :c_out] + b_ref[...]

    return pl.pallas_call(
        body, out_shape=jax.ShapeDtypeStruct((n, c_out), _F32)
    )(q0, q1, hs2p, dis_col, b2)


def kernel(x, edge_index, edge_attr, W1, b1, W2, b2):
    n, _dim = x.shape
    e = edge_attr.shape[0]
    n_pad = -(-n // (NS * ROWS_PER_TILE)) * (NS * ROWS_PER_TILE)

    align = NC * NS * CHUNK * 8
    e_pad = -(-e // align) * align
    pad = e_pad - e
    src = jnp.pad(edge_index[0], (0, pad)).reshape(-1, CHUNK)
    dst = jnp.pad(edge_index[1], (0, pad)).reshape(-1, CHUNK)
    ew = jnp.pad(edge_attr, (0, pad)).reshape(-1, CHUNK)

    deg_rows = e_pad // (NS * CHUNK)
    agg_rows = e_pad // (NC * NS * CHUNK)

    dis_pad = _make_sc_deg(n_pad, deg_rows)(dst, ew)
    dis_col = dis_pad[:n].reshape(n, 1)

    hs1 = _tc_scale(x, W1, dis_col, n_pad)
    p = _make_sc_agg(n_pad, W1.shape[1], agg_rows)(src, dst, ew, hs1)

    d2p = -(-W2.shape[1] // LANES) * LANES
    w2p = jnp.pad(W2, ((0, 0), (0, d2p - W2.shape[1])))
    hs2p = _tc_mid(p[0, :n], p[1, :n], hs1[:n], dis_col, b1, w2p, n_pad)

    q = _make_sc_agg(n_pad, d2p, agg_rows)(src, dst, ew, hs2p)
    return _tc_final(q[0, :n], q[1, :n], hs2p[:n], dis_col, b2)


# speedup vs baseline: 1.4798x; 1.1849x over previous
import functools

import jax
import jax.numpy as jnp
import numpy as np
from jax import lax
from jax.experimental import pallas as pl
from jax.experimental.pallas import tpu as pltpu
from jax.experimental.pallas import tpu_sc as plsc

NC = 2
NS = 16
LANES = 16
CHUNK = 128
ROWS_PER_TILE = 640
NBUF = 5
AHEAD = 4

_F32 = jnp.float32
_I32 = jnp.int32


def _zero_rows(zb, d):
    z = jnp.zeros((LANES,), _F32)

    def body(i, _):
        for k in range(d // LANES):
            zb[i, pl.ds(k * LANES, LANES)] = z
        return 0

    lax.fori_loop(0, 128, body, 0)


def _make_sc_deg(n_pad, n_chunk_rows):
    seg = n_pad // NS

    @functools.partial(
        pl.kernel,
        mesh=plsc.VectorSubcoreMesh(core_axis_name="c", subcore_axis_name="s"),
        out_type=jax.ShapeDtypeStruct((n_pad,), _F32),
        compiler_params=pltpu.CompilerParams(use_tc_tiling_on_sc=False),
        scratch_types=[
            pltpu.VMEM((n_chunk_rows, CHUNK), _I32),
            pltpu.VMEM((n_chunk_rows, CHUNK), _F32),
            pltpu.VMEM((seg,), _F32),
            pltpu.VMEM_SHARED((n_pad,), _F32),
        ],
    )
    def deg_kernel(dst2d, ew2d, dis_out, dst_all, ew_all, buf, acc):
        c = lax.axis_index("c")
        s = lax.axis_index("s")

        @pl.when(c == 0)
        def _():
            z = jnp.zeros((LANES,), _F32)

            def zb(i, _):
                buf[pl.ds(i * LANES, LANES)] = z
                return 0

            lax.fori_loop(0, seg // LANES, zb, 0)
            pltpu.sync_copy(buf, acc.at[pl.ds(s * seg, seg)])
            plsc.subcore_barrier()

            pltpu.sync_copy(dst2d.at[pl.ds(s * n_chunk_rows, n_chunk_rows)], dst_all)
            pltpu.sync_copy(ew2d.at[pl.ds(s * n_chunk_rows, n_chunk_rows)], ew_all)

            def chunk(g, _):
                pltpu.sync_copy(ew_all.at[g], acc.at[dst_all.at[g]], add=True)
                return 0

            lax.fori_loop(0, n_chunk_rows, chunk, 0)
            plsc.subcore_barrier()

            pltpu.sync_copy(acc.at[pl.ds(s * seg, seg)], buf)

            def newton(i, _):
                sl = pl.ds(i * LANES, LANES)
                d = buf[sl] + 1.0
                pos = d > 0.0
                bits = lax.bitcast_convert_type(d, _I32)
                y = lax.bitcast_convert_type(
                    jnp.int32(0x5F3759DF) - lax.shift_right_arithmetic(bits, 1), _F32
                )
                half = d * 0.5
                for _it in range(4):
                    y = y * (1.5 - half * y * y)
                buf[sl] = jnp.where(pos, y, 0.0)
                return 0

            lax.fori_loop(0, seg // LANES, newton, 0)
            pltpu.sync_copy(buf, dis_out.at[pl.ds(s * seg, seg)])

    return deg_kernel


def _make_sc_agg(n_pad, d, n_chunk_rows, bf16_in=False, col_split=False):
    seg = n_pad // NS
    in_dt = jnp.bfloat16 if bf16_in else _F32
    hs_rows = (NC * n_pad) if col_split else n_pad

    @functools.partial(
        pl.kernel,
        mesh=plsc.VectorSubcoreMesh(core_axis_name="c", subcore_axis_name="s"),
        out_type=pltpu.HBM((NC, n_pad, d), _F32),
        compiler_params=pltpu.CompilerParams(
            use_tc_tiling_on_sc=False, needs_layout_passes=False
        ),
        scratch_types=[
            pltpu.VMEM((n_chunk_rows, CHUNK), _I32),
            pltpu.VMEM((n_chunk_rows, CHUNK), _I32),
            pltpu.VMEM((n_chunk_rows, CHUNK), _F32),
            [pltpu.VMEM((CHUNK, d), in_dt) for _ in range(NBUF)],
            [pltpu.VMEM((CHUNK, d), _F32) for _ in range(NBUF if bf16_in else 0)],
            pltpu.VMEM((128, d), _F32),
            pltpu.VMEM_SHARED((n_pad, d), _F32),
            [pltpu.SemaphoreType.DMA for _ in range(NBUF)],
        ],
    )
    def agg_kernel(src2d, dst2d, ew2d, hs, out, src_all, dst_all, ew_all,
                   bufs, sbufs, zb, acc, sems):
        if not bf16_in:
            sbufs = bufs
        c = lax.axis_index("c")
        s = lax.axis_index("s")
        wid = s if col_split else c * NS + s

        _zero_rows(zb, d)
        for i in range(seg // 128):
            pltpu.sync_copy(zb, acc.at[pl.ds(s * seg + i * 128, 128)])
        plsc.subcore_barrier()

        pltpu.sync_copy(src2d.at[pl.ds(wid * n_chunk_rows, n_chunk_rows)], src_all)
        pltpu.sync_copy(dst2d.at[pl.ds(wid * n_chunk_rows, n_chunk_rows)], dst_all)
        pltpu.sync_copy(ew2d.at[pl.ds(wid * n_chunk_rows, n_chunk_rows)], ew_all)

        if col_split:
            off_v = jnp.full((LANES,), c * n_pad, _I32)

            def add_off(r, _):
                for kk in range(CHUNK // LANES):
                    sl = pl.ds(kk * LANES, LANES)
                    src_all[r, sl] = src_all[r, sl] + off_v
                return 0

            lax.fori_loop(0, n_chunk_rows, add_off, 0)

        n_rows = n_chunk_rows

        def gather(g, buf, sem):
            pltpu.async_copy(hs.at[src_all.at[g]], buf, sem)

        def gather_wait(g, buf, sem):
            pltpu.make_async_copy(hs.at[src_all.at[g]], buf, sem).wait()

        def scatter(g, sbuf, sem):
            pltpu.async_copy(sbuf, acc.at[dst_all.at[g]], sem, add=True)

        def scatter_wait(g, sbuf, sem):
            pltpu.make_async_copy(sbuf, acc.at[dst_all.at[g]], sem).wait()

        def scale(g, buf, sbuf):
            gv = jnp.full((LANES,), g, _I32)

            def body(j, _):
                cv = plsc.load_gather(ew_all, [gv, jnp.full((LANES,), j, _I32)])
                if bf16_in:
                    for k in range(d // (2 * LANES)):
                        v = buf[j, pl.ds(k * 2 * LANES, 2 * LANES)]
                        a, b = plsc.unpack(v, format=plsc.PackFormat.INTERLEAVED)
                        sbuf[j, pl.ds(k * 2 * LANES, LANES)] = a * cv
                        sbuf[j, pl.ds(k * 2 * LANES + LANES, LANES)] = b * cv
                else:
                    for k in range(d // LANES):
                        sl = pl.ds(k * LANES, LANES)
                        sbuf[j, sl] = buf[j, sl] * cv
                return 0

            lax.fori_loop(0, CHUNK, body, 0)

        for j in range(AHEAD):
            gather(j, bufs[j], sems[j])

        def rotation(gq, _):
            for j in range(NBUF):
                g = gq * NBUF + j
                buf, sbuf, sem = bufs[j], sbufs[j], sems[j]
                jn = (j + AHEAD) % NBUF
                gather_wait(g, buf, sem)
                scale(g, buf, sbuf)
                scatter(g, sbuf, sem)

                @pl.when(
                    jnp.logical_and(g + AHEAD < n_rows, g + AHEAD >= NBUF)
                )
                def _():
                    scatter_wait(g + AHEAD - NBUF, sbufs[jn], sems[jn])

                @pl.when(g + AHEAD < n_rows)
                def _():
                    gather(g + AHEAD, bufs[jn], sems[jn])
            return 0

        lax.fori_loop(0, n_rows // NBUF, rotation, 0)
        for j in range(NBUF):
            scatter_wait(n_rows - NBUF + j, sbufs[j], sems[j])
        plsc.subcore_barrier()

        for i in range(seg // 128):
            r0 = s * seg + i * 128
            pltpu.sync_copy(acc.at[pl.ds(r0, 128)], out.at[c, pl.ds(r0, 128)])

    return agg_kernel


def _tc_scale(x, w, wp_l, wp_r, dis_col, n_pad):
    n = x.shape[0]
    h = w.shape[1]
    hh = h // 2

    def body(x_ref, w_ref, wl_ref, wr_ref, d_ref, obf_ref, o_ref):
        hm = jnp.dot(x_ref[...], w_ref[...], preferred_element_type=_F32)
        hml = jnp.dot(x_ref[...], wl_ref[...], preferred_element_type=_F32)
        hmr = jnp.dot(x_ref[...], wr_ref[...], preferred_element_type=_F32)
        o_ref[...] = hm * d_ref[...]
        zpad = jnp.zeros((n_pad - n, hh), jnp.bfloat16)
        obf_ref[0, pl.ds(0, n), :] = (hml * d_ref[...]).astype(jnp.bfloat16)
        obf_ref[0, pl.ds(n, n_pad - n), :] = zpad
        obf_ref[1, pl.ds(0, n), :] = (hmr * d_ref[...]).astype(jnp.bfloat16)
        obf_ref[1, pl.ds(n, n_pad - n), :] = zpad

    return pl.pallas_call(
        body,
        out_shape=[
            jax.ShapeDtypeStruct((NC, n_pad, hh), jnp.bfloat16),
            jax.ShapeDtypeStruct((n, h), _F32),
        ],
    )(x, w, wp_l, wp_r, dis_col)


def _tc_mid(agg, hs1, dis_col, b1, w2p, n_pad):
    n, _h = hs1.shape
    d2 = w2p.shape[1]

    def body(p_ref, hs_ref, d_ref, b_ref, w_ref, o_ref):
        t = (p_ref[...] + hs_ref[...]) * d_ref[...] + b_ref[...]
        r = jnp.maximum(t, 0.0)
        o_ref[pl.ds(0, n), :] = (
            jnp.dot(r, w_ref[...], preferred_element_type=_F32) * d_ref[...]
        )
        o_ref[pl.ds(n, n_pad - n), :] = jnp.zeros((n_pad - n, d2), _F32)

    return pl.pallas_call(
        body, out_shape=jax.ShapeDtypeStruct((n_pad, d2), _F32)
    )(agg, hs1, dis_col, b1, w2p)


def _tc_final(q0, q1, hs2p, dis_col, b2):
    n = hs2p.shape[0]
    c_out = b2.shape[0]

    def body(q0_ref, q1_ref, hs_ref, d_ref, b_ref, o_ref):
        t = (q0_ref[...] + q1_ref[...] + hs_ref[...]) * d_ref[...]
        o_ref[...] = t[:, :c_out] + b_ref[...]

    return pl.pallas_call(
        body, out_shape=jax.ShapeDtypeStruct((n, c_out), _F32)
    )(q0, q1, hs2p, dis_col, b2)


def kernel(x, edge_index, edge_attr, W1, b1, W2, b2):
    n, _dim = x.shape
    e = edge_attr.shape[0]
    n_pad = -(-n // (NS * ROWS_PER_TILE)) * (NS * ROWS_PER_TILE)

    align = NC * NS * CHUNK * 8
    e_pad = -(-e // align) * align
    pad = e_pad - e
    src = jnp.pad(edge_index[0], (0, pad)).reshape(-1, CHUNK)
    dst = jnp.pad(edge_index[1], (0, pad)).reshape(-1, CHUNK)
    ew = jnp.pad(edge_attr, (0, pad)).reshape(-1, CHUNK)

    deg_rows = e_pad // (NS * CHUNK)
    agg_rows = e_pad // (NC * NS * CHUNK)

    dis_pad = _make_sc_deg(n_pad, deg_rows)(dst, ew)
    dis_col = dis_pad[:n].reshape(n, 1)

    h1 = W1.shape[1]
    pi = np.empty((h1,), np.int32)
    for k in range(h1 // 32):
        for pp in range(16):
            pi[32 * k + pp] = 32 * k + 2 * pp
            pi[32 * k + 16 + pp] = 32 * k + 2 * pp + 1
    pinv = np.empty((h1,), np.int32)
    pinv[pi] = np.arange(h1, dtype=np.int32)
    w1perm = W1[:, pinv]

    hs1_bf, hs1 = _tc_scale(
        x, W1, w1perm[:, : h1 // 2], w1perm[:, h1 // 2 :], dis_col, n_pad
    )
    split_rows = e_pad // (NS * CHUNK)
    p = _make_sc_agg(n_pad, h1 // 2, split_rows, bf16_in=True, col_split=True)(
        src, dst, ew, hs1_bf.reshape(NC * n_pad, h1 // 2)
    )
    agg1 = jnp.concatenate([p[0, :n], p[1, :n]], axis=1)

    d2p = -(-W2.shape[1] // LANES) * LANES
    w2p = jnp.pad(W2, ((0, 0), (0, d2p - W2.shape[1])))
    hs2p = _tc_mid(agg1, hs1, dis_col, b1, w2p, n_pad)

    q = _make_sc_agg(n_pad, d2p, agg_rows)(src, dst, ew, hs2p)
    return _tc_final(q[0, :n], q[1, :n], hs2p[:n], dis_col, b2)


# speedup vs baseline: 1.5882x; 1.0732x over previous
import functools

import jax
import jax.numpy as jnp
import numpy as np
from jax import lax
from jax.experimental import pallas as pl
from jax.experimental.pallas import tpu as pltpu
from jax.experimental.pallas import tpu_sc as plsc

NC = 2
NS = 16
LANES = 16
CHUNK = 128
ROWS_PER_TILE = 640
NBUF = 5
AHEAD = 4

_F32 = jnp.float32
_I32 = jnp.int32


def _zero_rows(zb, d):
    z = jnp.zeros((LANES,), _F32)

    def body(i, _):
        for k in range(d // LANES):
            zb[i, pl.ds(k * LANES, LANES)] = z
        return 0

    lax.fori_loop(0, 128, body, 0)


def _make_sc_deg(n_pad, n_chunk_rows):
    seg = n_pad // NS

    @functools.partial(
        pl.kernel,
        mesh=plsc.VectorSubcoreMesh(core_axis_name="c", subcore_axis_name="s"),
        out_type=jax.ShapeDtypeStruct((n_pad,), _F32),
        compiler_params=pltpu.CompilerParams(use_tc_tiling_on_sc=False),
        scratch_types=[
            pltpu.VMEM((n_chunk_rows, CHUNK), _I32),
            pltpu.VMEM((n_chunk_rows, CHUNK), _F32),
            pltpu.VMEM((seg,), _F32),
            pltpu.VMEM_SHARED((n_pad,), _F32),
        ],
    )
    def deg_kernel(dst2d, ew2d, dis_out, dst_all, ew_all, buf, acc):
        c = lax.axis_index("c")
        s = lax.axis_index("s")

        @pl.when(c == 0)
        def _():
            z = jnp.zeros((LANES,), _F32)

            def zb(i, _):
                buf[pl.ds(i * LANES, LANES)] = z
                return 0

            lax.fori_loop(0, seg // LANES, zb, 0)
            pltpu.sync_copy(buf, acc.at[pl.ds(s * seg, seg)])
            plsc.subcore_barrier()

            pltpu.sync_copy(dst2d.at[pl.ds(s * n_chunk_rows, n_chunk_rows)], dst_all)
            pltpu.sync_copy(ew2d.at[pl.ds(s * n_chunk_rows, n_chunk_rows)], ew_all)

            def chunk(g, _):
                pltpu.sync_copy(ew_all.at[g], acc.at[dst_all.at[g]], add=True)
                return 0

            lax.fori_loop(0, n_chunk_rows, chunk, 0)
            plsc.subcore_barrier()

            pltpu.sync_copy(acc.at[pl.ds(s * seg, seg)], buf)

            def newton(i, _):
                sl = pl.ds(i * LANES, LANES)
                d = buf[sl] + 1.0
                pos = d > 0.0
                bits = lax.bitcast_convert_type(d, _I32)
                y = lax.bitcast_convert_type(
                    jnp.int32(0x5F3759DF) - lax.shift_right_arithmetic(bits, 1), _F32
                )
                half = d * 0.5
                for _it in range(4):
                    y = y * (1.5 - half * y * y)
                buf[sl] = jnp.where(pos, y, 0.0)
                return 0

            lax.fori_loop(0, seg // LANES, newton, 0)
            pltpu.sync_copy(buf, dis_out.at[pl.ds(s * seg, seg)])

    return deg_kernel


def _make_sc_agg(n_pad, d, n_chunk_rows, bf16_in=False, col_split=False):
    seg = n_pad // NS
    in_dt = jnp.bfloat16 if bf16_in else _F32
    hs_rows = (NC * n_pad) if col_split else n_pad

    @functools.partial(
        pl.kernel,
        mesh=plsc.VectorSubcoreMesh(core_axis_name="c", subcore_axis_name="s"),
        out_type=pltpu.HBM((NC, n_pad, d), _F32),
        compiler_params=pltpu.CompilerParams(
            use_tc_tiling_on_sc=False, needs_layout_passes=False
        ),
        scratch_types=[
            pltpu.VMEM((n_chunk_rows, CHUNK), _I32),
            pltpu.VMEM((n_chunk_rows, CHUNK), _I32),
            pltpu.VMEM((n_chunk_rows, CHUNK), _F32),
            [pltpu.VMEM((CHUNK, d), in_dt) for _ in range(NBUF)],
            [pltpu.VMEM((CHUNK, d), _F32) for _ in range(NBUF if bf16_in else 0)],
            pltpu.VMEM((128, d), _F32),
            pltpu.VMEM_SHARED((n_pad, d), _F32),
            [pltpu.SemaphoreType.DMA for _ in range(NBUF)],
        ],
    )
    def agg_kernel(src2d, dst2d, ew2d, hs, out, src_all, dst_all, ew_all,
                   bufs, sbufs, zb, acc, sems):
        if not bf16_in:
            sbufs = bufs
        c = lax.axis_index("c")
        s = lax.axis_index("s")
        wid = s if col_split else c * NS + s

        _zero_rows(zb, d)
        for i in range(seg // 128):
            pltpu.sync_copy(zb, acc.at[pl.ds(s * seg + i * 128, 128)])
        plsc.subcore_barrier()

        pltpu.sync_copy(src2d.at[pl.ds(wid * n_chunk_rows, n_chunk_rows)], src_all)
        pltpu.sync_copy(dst2d.at[pl.ds(wid * n_chunk_rows, n_chunk_rows)], dst_all)
        pltpu.sync_copy(ew2d.at[pl.ds(wid * n_chunk_rows, n_chunk_rows)], ew_all)

        if col_split:
            off_v = jnp.full((LANES,), c * n_pad, _I32)

            def add_off(r, _):
                for kk in range(CHUNK // LANES):
                    sl = pl.ds(kk * LANES, LANES)
                    src_all[r, sl] = src_all[r, sl] + off_v
                return 0

            lax.fori_loop(0, n_chunk_rows, add_off, 0)

        n_rows = n_chunk_rows

        def gather(g, buf, sem):
            pltpu.async_copy(hs.at[src_all.at[g]], buf, sem)

        def gather_wait(g, buf, sem):
            pltpu.make_async_copy(hs.at[src_all.at[g]], buf, sem).wait()

        def scatter(g, sbuf, sem):
            pltpu.async_copy(sbuf, acc.at[dst_all.at[g]], sem, add=True)

        def scatter_wait(g, sbuf, sem):
            pltpu.make_async_copy(sbuf, acc.at[dst_all.at[g]], sem).wait()

        def scale(g, buf, sbuf):
            gv = jnp.full((LANES,), g, _I32)

            def body(j, _):
                cv = plsc.load_gather(ew_all, [gv, jnp.full((LANES,), j, _I32)])
                if bf16_in:
                    for k in range(d // (2 * LANES)):
                        v = buf[j, pl.ds(k * 2 * LANES, 2 * LANES)]
                        a, b = plsc.unpack(v, format=plsc.PackFormat.INTERLEAVED)
                        sbuf[j, pl.ds(k * 2 * LANES, LANES)] = a * cv
                        sbuf[j, pl.ds(k * 2 * LANES + LANES, LANES)] = b * cv
                else:
                    for k in range(d // LANES):
                        sl = pl.ds(k * LANES, LANES)
                        sbuf[j, sl] = buf[j, sl] * cv
                return 0

            lax.fori_loop(0, CHUNK, body, 0)

        for j in range(AHEAD):
            gather(j, bufs[j], sems[j])

        def rotation(gq, _):
            for j in range(NBUF):
                g = gq * NBUF + j
                buf, sbuf, sem = bufs[j], sbufs[j], sems[j]
                jn = (j + AHEAD) % NBUF
                gather_wait(g, buf, sem)
                scale(g, buf, sbuf)
                scatter(g, sbuf, sem)

                @pl.when(
                    jnp.logical_and(g + AHEAD < n_rows, g + AHEAD >= NBUF)
                )
                def _():
                    scatter_wait(g + AHEAD - NBUF, sbufs[jn], sems[jn])

                @pl.when(g + AHEAD < n_rows)
                def _():
                    gather(g + AHEAD, bufs[jn], sems[jn])
            return 0

        lax.fori_loop(0, n_rows // NBUF, rotation, 0)
        for j in range(NBUF):
            scatter_wait(n_rows - NBUF + j, sbufs[j], sems[j])
        plsc.subcore_barrier()

        for i in range(seg // 128):
            r0 = s * seg + i * 128
            pltpu.sync_copy(acc.at[pl.ds(r0, 128)], out.at[c, pl.ds(r0, 128)])

    return agg_kernel


def _make_sc_agg2(n_pad, n_chunk_rows):
    m = 2 * n_pad
    seg = m // NS

    @functools.partial(
        pl.kernel,
        mesh=plsc.VectorSubcoreMesh(core_axis_name="c", subcore_axis_name="s"),
        out_type=pltpu.HBM((NC, m), _F32),
        compiler_params=pltpu.CompilerParams(
            use_tc_tiling_on_sc=False, needs_layout_passes=False
        ),
        scratch_types=[
            pltpu.VMEM((n_chunk_rows, CHUNK), _I32),
            pltpu.VMEM((n_chunk_rows, CHUNK), _I32),
            pltpu.VMEM((n_chunk_rows, CHUNK), _F32),
            pltpu.VMEM((m,), _F32),
            pltpu.VMEM((seg,), _F32),
            [pltpu.VMEM((CHUNK,), _I32) for _ in range(2)],
            [pltpu.VMEM((CHUNK,), _F32) for _ in range(2)],
            pltpu.VMEM_SHARED((m,), _F32),
        ],
    )
    def agg2_kernel(src2d, dst2d, ew2d, hs2, out, src_all, dst_all, ew_all,
                    hsb, zb, ibufs, mbufs, acc):
        c = lax.axis_index("c")
        s = lax.axis_index("s")
        wid = c * NS + s

        z = jnp.zeros((LANES,), _F32)

        def zero(i, _):
            zb[pl.ds(i * LANES, LANES)] = z
            return 0

        lax.fori_loop(0, seg // LANES, zero, 0)
        pltpu.sync_copy(zb, acc.at[pl.ds(s * seg, seg)])

        pltpu.sync_copy(hs2, hsb)
        pltpu.sync_copy(src2d.at[pl.ds(wid * n_chunk_rows, n_chunk_rows)], src_all)
        pltpu.sync_copy(dst2d.at[pl.ds(wid * n_chunk_rows, n_chunk_rows)], dst_all)
        pltpu.sync_copy(ew2d.at[pl.ds(wid * n_chunk_rows, n_chunk_rows)], ew_all)
        plsc.subcore_barrier()

        def chunk(g, _):
            for blk in range(CHUNK // LANES):
                sl = pl.ds(blk * LANES, LANES)
                sv = src_all[g, sl] * 2
                dv = dst_all[g, sl] * 2
                wv = ew_all[g, sl]
                ibufs[0][sl] = dv
                ibufs[1][sl] = dv + 1
                mbufs[0][sl] = plsc.load_gather(hsb, [sv]) * wv
                mbufs[1][sl] = plsc.load_gather(hsb, [sv + 1]) * wv
            pltpu.sync_copy(mbufs[0], acc.at[ibufs[0]], add=True)
            pltpu.sync_copy(mbufs[1], acc.at[ibufs[1]], add=True)
            return 0

        lax.fori_loop(0, n_chunk_rows, chunk, 0)
        plsc.subcore_barrier()
        pltpu.sync_copy(acc.at[pl.ds(s * seg, seg)], zb)
        pltpu.sync_copy(zb, out.at[c, pl.ds(s * seg, seg)])

    return agg2_kernel


def _tc_scale(x, w, wp_l, wp_r, dis_col, n_pad):
    n = x.shape[0]
    h = w.shape[1]
    hh = h // 2

    def body(x_ref, w_ref, wl_ref, wr_ref, d_ref, obf_ref, o_ref):
        hm = jnp.dot(x_ref[...], w_ref[...], preferred_element_type=_F32)
        hml = jnp.dot(x_ref[...], wl_ref[...], preferred_element_type=_F32)
        hmr = jnp.dot(x_ref[...], wr_ref[...], preferred_element_type=_F32)
        o_ref[...] = hm * d_ref[...]
        zpad = jnp.zeros((n_pad - n, hh), jnp.bfloat16)
        obf_ref[0, pl.ds(0, n), :] = (hml * d_ref[...]).astype(jnp.bfloat16)
        obf_ref[0, pl.ds(n, n_pad - n), :] = zpad
        obf_ref[1, pl.ds(0, n), :] = (hmr * d_ref[...]).astype(jnp.bfloat16)
        obf_ref[1, pl.ds(n, n_pad - n), :] = zpad

    return pl.pallas_call(
        body,
        out_shape=[
            jax.ShapeDtypeStruct((NC, n_pad, hh), jnp.bfloat16),
            jax.ShapeDtypeStruct((n, h), _F32),
        ],
    )(x, w, wp_l, wp_r, dis_col)


def _tc_mid(agg, hs1, dis_col, b1, w2p, n_pad):
    n, _h = hs1.shape
    d2 = w2p.shape[1]

    def body(p_ref, hs_ref, d_ref, b_ref, w_ref, o_ref):
        t = (p_ref[...] + hs_ref[...]) * d_ref[...] + b_ref[...]
        r = jnp.maximum(t, 0.0)
        o_ref[pl.ds(0, n), :] = (
            jnp.dot(r, w_ref[...], preferred_element_type=_F32) * d_ref[...]
        )
        o_ref[pl.ds(n, n_pad - n), :] = jnp.zeros((n_pad - n, d2), _F32)

    return pl.pallas_call(
        body, out_shape=jax.ShapeDtypeStruct((n_pad, d2), _F32)
    )(agg, hs1, dis_col, b1, w2p)


def _tc_final(q0, q1, hs2p, dis_col, b2):
    n = hs2p.shape[0]
    c_out = b2.shape[0]

    def body(q0_ref, q1_ref, hs_ref, d_ref, b_ref, o_ref):
        t = (q0_ref[...] + q1_ref[...] + hs_ref[...]) * d_ref[...]
        o_ref[...] = t + b_ref[...]

    return pl.pallas_call(
        body, out_shape=jax.ShapeDtypeStruct((n, c_out), _F32)
    )(q0, q1, hs2p, dis_col, b2)


def kernel(x, edge_index, edge_attr, W1, b1, W2, b2):
    n, _dim = x.shape
    e = edge_attr.shape[0]
    n_pad = -(-n // (NS * ROWS_PER_TILE)) * (NS * ROWS_PER_TILE)

    align = NC * NS * CHUNK * 8
    e_pad = -(-e // align) * align
    pad = e_pad - e
    src = jnp.pad(edge_index[0], (0, pad)).reshape(-1, CHUNK)
    dst = jnp.pad(edge_index[1], (0, pad)).reshape(-1, CHUNK)
    ew = jnp.pad(edge_attr, (0, pad)).reshape(-1, CHUNK)

    deg_rows = e_pad // (NS * CHUNK)
    agg_rows = e_pad // (NC * NS * CHUNK)

    dis_pad = _make_sc_deg(n_pad, deg_rows)(dst, ew)
    dis_col = dis_pad[:n].reshape(n, 1)

    h1 = W1.shape[1]
    pi = np.empty((h1,), np.int32)
    for k in range(h1 // 32):
        for pp in range(16):
            pi[32 * k + pp] = 32 * k + 2 * pp
            pi[32 * k + 16 + pp] = 32 * k + 2 * pp + 1
    pinv = np.empty((h1,), np.int32)
    pinv[pi] = np.arange(h1, dtype=np.int32)
    w1perm = W1[:, pinv]

    hs1_bf, hs1 = _tc_scale(
        x, W1, w1perm[:, : h1 // 2], w1perm[:, h1 // 2 :], dis_col, n_pad
    )
    split_rows = e_pad // (NS * CHUNK)
    p = _make_sc_agg(n_pad, h1 // 2, split_rows, bf16_in=True, col_split=True)(
        src, dst, ew, hs1_bf.reshape(NC * n_pad, h1 // 2)
    )
    agg1 = jnp.concatenate([p[0, :n], p[1, :n]], axis=1)

    hs2p = _tc_mid(agg1, hs1, dis_col, b1, W2, n_pad)

    q = _make_sc_agg2(n_pad, deg_rows // NC)(
        src, dst, ew, hs2p.reshape(NC * n_pad * W2.shape[1] // 2)
    )
    q3 = q.reshape(NC, n_pad, W2.shape[1])
    return _tc_final(q3[0, :n], q3[1, :n], hs2p[:n], dis_col, b2)


# speedup vs baseline: 1.6357x; 1.0299x over previous
import functools

import jax
import jax.numpy as jnp
import numpy as np
from jax import lax
from jax.experimental import pallas as pl
from jax.experimental.pallas import tpu as pltpu
from jax.experimental.pallas import tpu_sc as plsc

NC = 2
NS = 16
LANES = 16
CHUNK = 128
ROWS_PER_TILE = 640
NBUF = 5
AHEAD = 4

_F32 = jnp.float32
_I32 = jnp.int32


def _zero_rows(zb, d):
    z = jnp.zeros((LANES,), _F32)

    def body(i, _):
        for k in range(d // LANES):
            zb[i, pl.ds(k * LANES, LANES)] = z
        return 0

    lax.fori_loop(0, 128, body, 0)


def _make_sc_deg(n_pad, n_chunk_rows):
    seg = n_pad // NS

    @functools.partial(
        pl.kernel,
        mesh=plsc.VectorSubcoreMesh(core_axis_name="c", subcore_axis_name="s"),
        out_type=jax.ShapeDtypeStruct((n_pad,), _F32),
        compiler_params=pltpu.CompilerParams(use_tc_tiling_on_sc=False),
        scratch_types=[
            pltpu.VMEM((n_chunk_rows, CHUNK), _I32),
            pltpu.VMEM((n_chunk_rows, CHUNK), _F32),
            pltpu.VMEM((seg,), _F32),
            pltpu.VMEM_SHARED((n_pad,), _F32),
        ],
    )
    def deg_kernel(dst2d, ew2d, dis_out, dst_all, ew_all, buf, acc):
        c = lax.axis_index("c")
        s = lax.axis_index("s")

        @pl.when(c == 0)
        def _():
            z = jnp.zeros((LANES,), _F32)

            def zb(i, _):
                buf[pl.ds(i * LANES, LANES)] = z
                return 0

            lax.fori_loop(0, seg // LANES, zb, 0)
            pltpu.sync_copy(buf, acc.at[pl.ds(s * seg, seg)])
            plsc.subcore_barrier()

            pltpu.sync_copy(dst2d.at[pl.ds(s * n_chunk_rows, n_chunk_rows)], dst_all)
            pltpu.sync_copy(ew2d.at[pl.ds(s * n_chunk_rows, n_chunk_rows)], ew_all)

            def chunk(g, _):
                pltpu.sync_copy(ew_all.at[g], acc.at[dst_all.at[g]], add=True)
                return 0

            lax.fori_loop(0, n_chunk_rows, chunk, 0)
            plsc.subcore_barrier()

            pltpu.sync_copy(acc.at[pl.ds(s * seg, seg)], buf)

            def newton(i, _):
                sl = pl.ds(i * LANES, LANES)
                d = buf[sl] + 1.0
                pos = d > 0.0
                bits = lax.bitcast_convert_type(d, _I32)
                y = lax.bitcast_convert_type(
                    jnp.int32(0x5F3759DF) - lax.shift_right_arithmetic(bits, 1), _F32
                )
                half = d * 0.5
                for _it in range(4):
                    y = y * (1.5 - half * y * y)
                buf[sl] = jnp.where(pos, y, 0.0)
                return 0

            lax.fori_loop(0, seg // LANES, newton, 0)
            pltpu.sync_copy(buf, dis_out.at[pl.ds(s * seg, seg)])

    return deg_kernel


def _make_sc_agg(n_pad, d, n_chunk_rows, bf16_in=False, col_split=False):
    seg = n_pad // NS
    in_dt = jnp.bfloat16 if bf16_in else _F32
    hs_rows = (NC * n_pad) if col_split else n_pad

    @functools.partial(
        pl.kernel,
        mesh=plsc.VectorSubcoreMesh(core_axis_name="c", subcore_axis_name="s"),
        out_type=pltpu.HBM((NC, n_pad, d), _F32),
        compiler_params=pltpu.CompilerParams(
            use_tc_tiling_on_sc=False, needs_layout_passes=False
        ),
        scratch_types=[
            pltpu.VMEM((n_chunk_rows, CHUNK), _I32),
            pltpu.VMEM((n_chunk_rows, CHUNK), _I32),
            pltpu.VMEM((n_chunk_rows, CHUNK), _F32),
            [pltpu.VMEM((CHUNK, d), in_dt) for _ in range(NBUF)],
            [pltpu.VMEM((CHUNK, d), _F32) for _ in range(NBUF if bf16_in else 0)],
            pltpu.VMEM((128, d), _F32),
            pltpu.VMEM_SHARED((n_pad, d), _F32),
            [pltpu.SemaphoreType.DMA for _ in range(NBUF)],
        ],
    )
    def agg_kernel(src2d, dst2d, ew2d, hs, out, src_all, dst_all, ew_all,
                   bufs, sbufs, zb, acc, sems):
        if not bf16_in:
            sbufs = bufs
        c = lax.axis_index("c")
        s = lax.axis_index("s")
        wid = s if col_split else c * NS + s

        _zero_rows(zb, d)
        for i in range(seg // 128):
            pltpu.sync_copy(zb, acc.at[pl.ds(s * seg + i * 128, 128)])
        plsc.subcore_barrier()

        pltpu.sync_copy(src2d.at[pl.ds(wid * n_chunk_rows, n_chunk_rows)], src_all)
        pltpu.sync_copy(dst2d.at[pl.ds(wid * n_chunk_rows, n_chunk_rows)], dst_all)
        pltpu.sync_copy(ew2d.at[pl.ds(wid * n_chunk_rows, n_chunk_rows)], ew_all)

        if col_split:
            off_v = jnp.full((LANES,), c * n_pad, _I32)

            def add_off(r, _):
                for kk in range(CHUNK // LANES):
                    sl = pl.ds(kk * LANES, LANES)
                    src_all[r, sl] = src_all[r, sl] + off_v
                return 0

            lax.fori_loop(0, n_chunk_rows, add_off, 0)

        n_rows = n_chunk_rows

        def gather(g, buf, sem):
            pltpu.async_copy(hs.at[src_all.at[g]], buf, sem)

        def gather_wait(g, buf, sem):
            pltpu.make_async_copy(hs.at[src_all.at[g]], buf, sem).wait()

        def scatter(g, sbuf, sem):
            pltpu.async_copy(sbuf, acc.at[dst_all.at[g]], sem, add=True)

        def scatter_wait(g, sbuf, sem):
            pltpu.make_async_copy(sbuf, acc.at[dst_all.at[g]], sem).wait()

        def scale(g, buf, sbuf):
            gv = jnp.full((LANES,), g, _I32)

            def one_row(j):
                cv = plsc.load_gather(ew_all, [gv, jnp.full((LANES,), j, _I32)])
                if bf16_in:
                    for k in range(d // (2 * LANES)):
                        v = buf[j, pl.ds(k * 2 * LANES, 2 * LANES)]
                        a, b = plsc.unpack(v, format=plsc.PackFormat.INTERLEAVED)
                        sbuf[j, pl.ds(k * 2 * LANES, LANES)] = a * cv
                        sbuf[j, pl.ds(k * 2 * LANES + LANES, LANES)] = b * cv
                else:
                    for k in range(d // LANES):
                        sl = pl.ds(k * LANES, LANES)
                        sbuf[j, sl] = buf[j, sl] * cv

            def body(jq, _):
                for r in range(4):
                    one_row(jq * 4 + r)
                return 0

            lax.fori_loop(0, CHUNK // 4, body, 0)

        for j in range(AHEAD):
            gather(j, bufs[j], sems[j])

        def rotation(gq, _):
            for j in range(NBUF):
                g = gq * NBUF + j
                buf, sbuf, sem = bufs[j], sbufs[j], sems[j]
                jn = (j + AHEAD) % NBUF
                gather_wait(g, buf, sem)
                scale(g, buf, sbuf)
                scatter(g, sbuf, sem)

                @pl.when(
                    jnp.logical_and(g + AHEAD < n_rows, g + AHEAD >= NBUF)
                )
                def _():
                    scatter_wait(g + AHEAD - NBUF, sbufs[jn], sems[jn])

                @pl.when(g + AHEAD < n_rows)
                def _():
                    gather(g + AHEAD, bufs[jn], sems[jn])
            return 0

        lax.fori_loop(0, n_rows // NBUF, rotation, 0)
        for j in range(NBUF):
            scatter_wait(n_rows - NBUF + j, sbufs[j], sems[j])
        plsc.subcore_barrier()

        for i in range(seg // 128):
            r0 = s * seg + i * 128
            pltpu.sync_copy(acc.at[pl.ds(r0, 128)], out.at[c, pl.ds(r0, 128)])

    return agg_kernel


def _make_sc_agg2(n_pad, n_chunk_rows):
    m = 2 * n_pad
    seg = m // NS

    @functools.partial(
        pl.kernel,
        mesh=plsc.VectorSubcoreMesh(core_axis_name="c", subcore_axis_name="s"),
        out_type=pltpu.HBM((NC, m), _F32),
        compiler_params=pltpu.CompilerParams(
            use_tc_tiling_on_sc=False, needs_layout_passes=False
        ),
        scratch_types=[
            pltpu.VMEM((n_chunk_rows, CHUNK), _I32),
            pltpu.VMEM((n_chunk_rows, CHUNK), _I32),
            pltpu.VMEM((n_chunk_rows, CHUNK), _F32),
            pltpu.VMEM((m,), _F32),
            pltpu.VMEM((seg,), _F32),
            [pltpu.VMEM((CHUNK,), _I32) for _ in range(2)],
            [pltpu.VMEM((CHUNK,), _F32) for _ in range(2)],
            pltpu.VMEM_SHARED((m,), _F32),
        ],
    )
    def agg2_kernel(src2d, dst2d, ew2d, hs2, out, src_all, dst_all, ew_all,
                    hsb, zb, ibufs, mbufs, acc):
        c = lax.axis_index("c")
        s = lax.axis_index("s")
        wid = c * NS + s

        z = jnp.zeros((LANES,), _F32)

        def zero(i, _):
            zb[pl.ds(i * LANES, LANES)] = z
            return 0

        lax.fori_loop(0, seg // LANES, zero, 0)
        pltpu.sync_copy(zb, acc.at[pl.ds(s * seg, seg)])

        pltpu.sync_copy(hs2, hsb)
        pltpu.sync_copy(src2d.at[pl.ds(wid * n_chunk_rows, n_chunk_rows)], src_all)
        pltpu.sync_copy(dst2d.at[pl.ds(wid * n_chunk_rows, n_chunk_rows)], dst_all)
        pltpu.sync_copy(ew2d.at[pl.ds(wid * n_chunk_rows, n_chunk_rows)], ew_all)
        plsc.subcore_barrier()

        def chunk(g, _):
            for blk in range(CHUNK // LANES):
                sl = pl.ds(blk * LANES, LANES)
                sv = src_all[g, sl] * 2
                dv = dst_all[g, sl] * 2
                wv = ew_all[g, sl]
                ibufs[0][sl] = dv
                ibufs[1][sl] = dv + 1
                mbufs[0][sl] = plsc.load_gather(hsb, [sv]) * wv
                mbufs[1][sl] = plsc.load_gather(hsb, [sv + 1]) * wv
            pltpu.sync_copy(mbufs[0], acc.at[ibufs[0]], add=True)
            pltpu.sync_copy(mbufs[1], acc.at[ibufs[1]], add=True)
            return 0

        lax.fori_loop(0, n_chunk_rows, chunk, 0)
        plsc.subcore_barrier()
        pltpu.sync_copy(acc.at[pl.ds(s * seg, seg)], zb)
        pltpu.sync_copy(zb, out.at[c, pl.ds(s * seg, seg)])

    return agg2_kernel


def _tc_scale(x, w, wp_l, wp_r, dis_col, n_pad):
    n = x.shape[0]
    h = w.shape[1]
    hh = h // 2

    def body(x_ref, w_ref, wl_ref, wr_ref, d_ref, obf_ref, o_ref):
        hm = jnp.dot(x_ref[...], w_ref[...], preferred_element_type=_F32)
        hml = jnp.dot(x_ref[...], wl_ref[...], preferred_element_type=_F32)
        hmr = jnp.dot(x_ref[...], wr_ref[...], preferred_element_type=_F32)
        o_ref[...] = hm * d_ref[...]
        zpad = jnp.zeros((n_pad - n, hh), jnp.bfloat16)
        obf_ref[0, pl.ds(0, n), :] = (hml * d_ref[...]).astype(jnp.bfloat16)
        obf_ref[0, pl.ds(n, n_pad - n), :] = zpad
        obf_ref[1, pl.ds(0, n), :] = (hmr * d_ref[...]).astype(jnp.bfloat16)
        obf_ref[1, pl.ds(n, n_pad - n), :] = zpad

    return pl.pallas_call(
        body,
        out_shape=[
            jax.ShapeDtypeStruct((NC, n_pad, hh), jnp.bfloat16),
            jax.ShapeDtypeStruct((n, h), _F32),
        ],
    )(x, w, wp_l, wp_r, dis_col)


def _tc_mid(agg, hs1, dis_col, b1, w2p, n_pad):
    n, _h = hs1.shape
    d2 = w2p.shape[1]

    def body(p_ref, hs_ref, d_ref, b_ref, w_ref, o_ref):
        t = (p_ref[...] + hs_ref[...]) * d_ref[...] + b_ref[...]
        r = jnp.maximum(t, 0.0)
        o_ref[pl.ds(0, n), :] = (
            jnp.dot(r, w_ref[...], preferred_element_type=_F32) * d_ref[...]
        )
        o_ref[pl.ds(n, n_pad - n), :] = jnp.zeros((n_pad - n, d2), _F32)

    return pl.pallas_call(
        body, out_shape=jax.ShapeDtypeStruct((n_pad, d2), _F32)
    )(agg, hs1, dis_col, b1, w2p)


def _tc_final(q0, q1, hs2p, dis_col, b2):
    n = hs2p.shape[0]
    c_out = b2.shape[0]

    def body(q0_ref, q1_ref, hs_ref, d_ref, b_ref, o_ref):
        t = (q0_ref[...] + q1_ref[...] + hs_ref[...]) * d_ref[...]
        o_ref[...] = t + b_ref[...]

    return pl.pallas_call(
        body, out_shape=jax.ShapeDtypeStruct((n, c_out), _F32)
    )(q0, q1, hs2p, dis_col, b2)


def kernel(x, edge_index, edge_attr, W1, b1, W2, b2):
    n, _dim = x.shape
    e = edge_attr.shape[0]
    n_pad = -(-n // (NS * ROWS_PER_TILE)) * (NS * ROWS_PER_TILE)

    align = NC * NS * CHUNK * 8
    e_pad = -(-e // align) * align
    pad = e_pad - e
    src = jnp.pad(edge_index[0], (0, pad)).reshape(-1, CHUNK)
    dst = jnp.pad(edge_index[1], (0, pad)).reshape(-1, CHUNK)
    ew = jnp.pad(edge_attr, (0, pad)).reshape(-1, CHUNK)

    deg_rows = e_pad // (NS * CHUNK)
    agg_rows = e_pad // (NC * NS * CHUNK)

    dis_pad = _make_sc_deg(n_pad, deg_rows)(dst, ew)
    dis_col = dis_pad[:n].reshape(n, 1)

    h1 = W1.shape[1]
    pi = np.empty((h1,), np.int32)
    for k in range(h1 // 32):
        for pp in range(16):
            pi[32 * k + pp] = 32 * k + 2 * pp
            pi[32 * k + 16 + pp] = 32 * k + 2 * pp + 1
    pinv = np.empty((h1,), np.int32)
    pinv[pi] = np.arange(h1, dtype=np.int32)
    w1perm = W1[:, pinv]

    hs1_bf, hs1 = _tc_scale(
        x, W1, w1perm[:, : h1 // 2], w1perm[:, h1 // 2 :], dis_col, n_pad
    )
    split_rows = e_pad // (NS * CHUNK)
    p = _make_sc_agg(n_pad, h1 // 2, split_rows, bf16_in=True, col_split=True)(
        src, dst, ew, hs1_bf.reshape(NC * n_pad, h1 // 2)
    )
    agg1 = jnp.concatenate([p[0, :n], p[1, :n]], axis=1)

    hs2p = _tc_mid(agg1, hs1, dis_col, b1, W2, n_pad)

    q = _make_sc_agg2(n_pad, deg_rows // NC)(
        src, dst, ew, hs2p.reshape(NC * n_pad * W2.shape[1] // 2)
    )
    q3 = q.reshape(NC, n_pad, W2.shape[1])
    return _tc_final(q3[0, :n], q3[1, :n], hs2p[:n], dis_col, b2)
